# Initial kernel scaffold; baseline (speedup 1.0000x reference)
import jax
import jax.numpy as jnp
from jax.experimental import pallas as pl


def kernel(x_nchw, w1, b1, w2, b2, w3, b3, fc1_w, fc1_b, fc2_wt, fc2_b):
    raise NotImplementedError("write your pallas kernel here")


# speedup vs baseline: 9.7050x; 9.7050x over previous
import jax
import jax.numpy as jnp
from jax.experimental import pallas as pl
from jax.experimental.pallas import tpu as pltpu

_VMEM_LIMIT = 32 * 1024 * 1024

H0 = 96
H1, W1 = 48, 16
H2, W2 = 24, 16


def _taps(flat, w_ref, b_ref, *, W, HW):
    acc = jnp.dot(flat[0:HW, :], w_ref[0], preferred_element_type=jnp.float32)
    acc += jnp.dot(flat[W:W + HW, :], w_ref[1],
                   preferred_element_type=jnp.float32)
    acc += jnp.dot(flat[2 * W:2 * W + HW, :], w_ref[2],
                   preferred_element_type=jnp.float32)
    return jnp.maximum(acc + b_ref[...], 0.0).astype(jnp.bfloat16)


def _hsplit(y, *, H, W, C):
    v = y.reshape(H // 2, 2, W, C)
    return v[:, 0], v[:, 1]


def _shift_down(z):
    zero = jnp.zeros_like(z[:, :1, :])
    return jnp.concatenate([zero, z[:, :-1, :]], axis=1)


def _shift_up(z):
    zero = jnp.zeros_like(z[:, :1, :])
    return jnp.concatenate([z[:, 1:, :], zero], axis=1)


def _hpad_flat(z, *, H, W, C):
    zh = jnp.zeros((1, W, C), jnp.bfloat16)
    return jnp.concatenate([zh, z, zh], axis=0).reshape((H + 2) * W, C)


def _fused_conv_kernel(x_ref, w1_ref, b1_ref, w2_ref, b2_ref, w3_ref, b3_ref,
                       o_ref):
    yg = [
        _taps(x_ref[0, g], w1_ref, b1_ref, W=16, HW=H0 * 16) for g in range(4)
    ]
    p = [_hsplit(y, H=H0, W=16, C=16) for y in yg]
    z_e = jnp.concatenate([p[0][0], p[2][0], p[0][1], p[2][1]], axis=-1)
    z_o = jnp.concatenate([p[1][0], p[3][0], p[1][1], p[3][1]], axis=-1)

    f2e = jnp.concatenate([_shift_down(z_o), z_e, z_o], axis=-1)
    f2o = jnp.concatenate([z_e, z_o, _shift_up(z_e)], axis=-1)
    y2e = _taps(_hpad_flat(f2e, H=H1, W=W1, C=192), w2_ref, b2_ref,
                W=W1, HW=H1 * W1)
    y2o = _taps(_hpad_flat(f2o, H=H1, W=W1, C=192), w2_ref, b2_ref,
                W=W1, HW=H1 * W1)

    qe = _hsplit(y2e, H=H1, W=W1, C=32)
    qo = _hsplit(y2o, H=H1, W=W1, C=32)
    z3 = jnp.concatenate([qe[0], qo[0], qe[1], qo[1]], axis=-1)

    zero = jnp.zeros((H2, 1, 128), jnp.bfloat16)
    zw = jnp.concatenate([zero, z3, zero], axis=1)
    f3 = jnp.concatenate([zw[:, k:k + W2, :] for k in range(3)], axis=-1)
    y3 = _taps(_hpad_flat(f3, H=H2, W=W2, C=384), w3_ref, b3_ref,
               W=W2, HW=H2 * W2)
    o_ref[0] = y3


def _conv_stack(xw, w1r, b1, w2r, b2, w3r, b3):
    B = xw.shape[0]
    rows = xw.shape[2]
    return pl.pallas_call(
        _fused_conv_kernel,
        out_shape=jax.ShapeDtypeStruct((B, H2 * W2, 64), jnp.bfloat16),
        grid_spec=pltpu.PrefetchScalarGridSpec(
            num_scalar_prefetch=0,
            grid=(B,),
            in_specs=[
                pl.BlockSpec((1, 4, rows, 9), lambda i: (i, 0, 0, 0)),
                pl.BlockSpec((3, 9, 16), lambda i: (0, 0, 0)),
                pl.BlockSpec((1, 16), lambda i: (0, 0)),
                pl.BlockSpec((3, 192, 32), lambda i: (0, 0, 0)),
                pl.BlockSpec((1, 32), lambda i: (0, 0)),
                pl.BlockSpec((3, 384, 64), lambda i: (0, 0, 0)),
                pl.BlockSpec((1, 64), lambda i: (0, 0)),
            ],
            out_specs=pl.BlockSpec((1, H2 * W2, 64), lambda i: (i, 0, 0)),
        ),
        compiler_params=pltpu.CompilerParams(
            dimension_semantics=("parallel",),
            vmem_limit_bytes=_VMEM_LIMIT,
        ),
    )(xw, w1r, b1, w2r, b2, w3r, b3)


def _mlp_kernel(x_ref, w1_ref, b1_ref, w2t_ref, b2_ref, o_ref):
    h = jnp.dot(x_ref[...], w1_ref[...], preferred_element_type=jnp.float32)
    h = jnp.maximum(h + b1_ref[...], 0.0)
    y = jnp.sum(h * w2t_ref[...], axis=1, keepdims=True) + b2_ref[...]
    o_ref[...] = y.astype(o_ref.dtype)


def _mlp_head(x, w1, b1, w2t, b2):
    bsz, k = x.shape
    n = w1.shape[-1]
    mb = 128 if bsz % 128 == 0 else bsz
    return pl.pallas_call(
        _mlp_kernel,
        out_shape=jax.ShapeDtypeStruct((bsz, 1), jnp.float32),
        grid_spec=pltpu.PrefetchScalarGridSpec(
            num_scalar_prefetch=0,
            grid=(bsz // mb,),
            in_specs=[
                pl.BlockSpec((mb, k), lambda i: (i, 0)),
                pl.BlockSpec((k, n), lambda i: (0, 0)),
                pl.BlockSpec((1, n), lambda i: (0, 0)),
                pl.BlockSpec((1, n), lambda i: (0, 0)),
                pl.BlockSpec((1, 1), lambda i: (0, 0)),
            ],
            out_specs=pl.BlockSpec((mb, 1), lambda i: (i, 0)),
        ),
        compiler_params=pltpu.CompilerParams(
            dimension_semantics=("parallel",),
            vmem_limit_bytes=_VMEM_LIMIT,
        ),
    )(x, w1, b1, w2t, b2)


def _permute_s2d_weight(w, cin_base, cout):
    wt = w.reshape(3, 3, cin_base, 2, 2, cout)
    wt = jnp.transpose(wt, (0, 1, 3, 4, 2, 5))
    return wt.reshape(3, 3 * 4 * cin_base, cout)


def kernel(x_nchw, w1, b1, w2, b2, w3, b3, fc1_w, fc1_b, fc2_wt, fc2_b):
    B = x_nchw.shape[0]

    x = jnp.transpose(x_nchw, (0, 2, 3, 1)).astype(jnp.bfloat16)
    xp = jnp.pad(x, ((0, 0), (1, 1), (1, 1), (0, 0)))
    t = jnp.concatenate([xp[:, :, k:k + 64, :] for k in range(3)], axis=-1)
    t = t.reshape(B, 98, 16, 2, 2, 9)
    t = jnp.transpose(t, (0, 4, 3, 1, 2, 5))
    xw = t.reshape(B, 4, 98 * 16, 9)

    w1r = w1.reshape(3, 9, 16)
    w2r = _permute_s2d_weight(w2, 16, 32)
    w3r = _permute_s2d_weight(w3, 32, 64)

    y3 = _conv_stack(xw, w1r, b1, w2r, b2, w3r, b3)
    xf = y3.reshape(B, H2 * W2 * 64)
    return _mlp_head(xf, fc1_w, fc1_b, fc2_wt, fc2_b)
